# baseline (speedup 1.0000x reference)
import jax
import jax.numpy as jnp
from jax import lax
from jax.experimental import pallas as pl
from jax.experimental.pallas import tpu as pltpu
from jax.experimental.pallas import tpu_sc as plsc

N = 10000
ND = 128
M = 480000
H = 64

R = 128
NCH = M // R
NW = 32
FPAD = 30720
EPAD = 10240

DB = 1000


def _dense_body(x_ref, w1_ref, b1_ref, w2_ref, b2_ref, w3t_ref, b3_ref,
                e_ref, g_ref):
    x = x_ref[...]
    w1 = w1_ref[...]
    w2 = w2_ref[...]
    w3t = w3t_ref[...]
    z1 = jnp.dot(x, w1, preferred_element_type=jnp.float32) + b1_ref[...]
    h1 = jnp.maximum(z1, 0.0)
    z2 = jnp.dot(h1, w2, preferred_element_type=jnp.float32) + b2_ref[...]
    h2 = jnp.maximum(z2, 0.0)
    h2r = h2.astype(jnp.bfloat16).astype(jnp.float32)
    w3b = jnp.broadcast_to(w3t, h2.shape
                           ).astype(jnp.bfloat16).astype(jnp.float32)
    e = jnp.sum(h2r * w3b, axis=1, keepdims=True) + b3_ref[0, 0]
    e_ref[...] = e
    g2 = jnp.where(z2 > 0, jnp.broadcast_to(w3t, z2.shape), 0.0)
    g1p = lax.dot_general(g2, w2, (((1,), (1,)), ((), ())),
                          preferred_element_type=jnp.float32)
    g1 = jnp.where(z1 > 0, g1p, 0.0)
    g_ref[...] = lax.dot_general(g1, w1, (((1,), (1,)), ((), ())),
                                 preferred_element_type=jnp.float32)


def _dense(x, W1, b1, W2, b2, W3, b3):
    w3t = W3.reshape(1, H)
    b1r = b1.reshape(1, H)
    b2r = b2.reshape(1, H)
    b3r = b3.reshape(1, 1)
    grid = N // DB
    full = lambda shape: pl.BlockSpec(shape, lambda i: (0, 0))
    return pl.pallas_call(
        _dense_body,
        grid=(grid,),
        in_specs=[
            pl.BlockSpec((DB, ND), lambda i: (i, 0)),
            full((ND, H)), full((1, H)),
            full((H, H)), full((1, H)),
            full((1, H)), full((1, 1)),
        ],
        out_specs=[
            pl.BlockSpec((DB, 1), lambda i: (i, 0)),
            pl.BlockSpec((DB, ND), lambda i: (i, 0)),
        ],
        out_shape=[
            jax.ShapeDtypeStruct((N, 1), jnp.float32),
            jax.ShapeDtypeStruct((N, ND), jnp.float32),
        ],
    )(x, W1, b1r, W2, b2r, w3t, b3r)


BLK = 16
IDXPAD = BLK * R


def _sc_body(xd_hbm, c0_hbm, uj_hbm, c2_hbm, ded_hbm, pae_hbm, idx_hbm,
             fpart_hbm, eout_hbm,
             xdv, dedv, c0b, ujb, c2b, facc, eacc, ebuf, ibuf,
             sem_a, sem_b, sem_i):
    nc = 2
    wid = lax.axis_index("s") * nc + lax.axis_index("c")
    lanes = lax.iota(jnp.int32, 16)

    def zf(i, _):
        for u in range(8):
            facc[pl.ds(i * 128 + u * 16, 16)] = jnp.zeros((16,), jnp.float32)
        return 0
    lax.fori_loop(0, FPAD // 128, zf, 0)

    @pl.when(wid == 0)
    def _energy():
        def ze(i, _):
            for u in range(8):
                eacc[pl.ds(i * 128 + u * 16, 16)] = jnp.zeros((16,),
                                                              jnp.float32)
            return 0
        lax.fori_loop(0, EPAD // 128, ze, 0)

        def echunk(c, _):
            pltpu.sync_copy(pae_hbm.at[pl.ds(c * 2000, 2000)],
                            ebuf.at[pl.ds(0, 2000)])
            pltpu.sync_copy(idx_hbm.at[pl.ds(c * 2000, 2000)],
                            ibuf.at[pl.ds(0, 2000)])

            def ebody(i, _):
                ev = ebuf[pl.ds(i * 16, 16)]
                iv = ibuf[pl.ds(i * 16, 16)]
                plsc.addupdate_scatter(eacc, [iv], ev)
                return 0
            lax.fori_loop(0, 125, ebody, 0)
            return 0
        lax.fori_loop(0, 5, echunk, 0)
        pltpu.sync_copy(eacc.at[pl.ds(0, N)], eout_hbm)

    n_i = jnp.where(wid == 0, 110, 117 + jnp.where(wid < 14, 1, 0))
    start = jnp.where(wid == 0, 0,
                      110 + (wid - 1) * 117 + jnp.minimum(wid - 1, 13))
    n_b = (n_i + BLK - 1) // BLK

    offs = [(lanes + t) & 15 for t in range(16)]
    zero16 = jnp.zeros((16,), jnp.float32)

    def _idx_descs(b, bp, sem):
        base = jnp.minimum((start + b * BLK) * R, M - IDXPAD)
        vb = bp * IDXPAD
        return (
            pltpu.make_async_copy(c0_hbm.at[pl.ds(base, IDXPAD)],
                                  c0b.at[pl.ds(vb, IDXPAD)], sem),
            pltpu.make_async_copy(uj_hbm.at[pl.ds(base, IDXPAD)],
                                  ujb.at[pl.ds(vb, IDXPAD)], sem),
            pltpu.make_async_copy(c2_hbm.at[pl.ds(base, IDXPAD)],
                                  c2b.at[pl.ds(vb, IDXPAD)], sem),
        )

    def _xd_desc(base_rows, k, p, sem):
        return pltpu.make_async_copy(
            xd_hbm.at[pl.ds(base_rows + k * R, R)],
            xdv.at[pl.ds(p * R, R)], sem)

    def _g_desc(bp, koff, k, p, sem):
        return pltpu.make_async_copy(
            ded_hbm.at[c0b.at[pl.ds(bp * IDXPAD + koff + k * R, R)]],
            dedv.at[pl.ds(p * R, R)], sem)

    for d in _idx_descs(0, 0, sem_a):
        d.start()
        d.wait()

    @pl.when(n_b > 1)
    def _():
        for d in _idx_descs(1, 1, sem_i):
            d.start()

    def block_body(b, _):
        j0 = b * BLK
        jn = jnp.minimum(BLK, n_i - j0)
        bp = b % 2
        base_rows = (start + j0) * R
        koff = base_rows - jnp.minimum(base_rows, M - IDXPAD)

        def issue(k, p, sem):
            _xd_desc(base_rows, k, p, sem).start()
            _g_desc(bp, koff, k, p, sem).start()

        def wait(k, p, sem):
            _xd_desc(base_rows, k, p, sem).wait()
            _g_desc(bp, koff, k, p, sem).wait()

        def compute(k, p):
            def rows16(r16, _):
                row = p * R + r16 * 16 + lanes

                def dcol(j, accs):
                    a0, a1, a2, a3 = accs
                    dvec = jnp.full((16,), j * 16, jnp.int32)
                    for t in range(16):
                        col = offs[t] + dvec
                        a = plsc.load_gather(xdv, [row, col])
                        bb = plsc.load_gather(dedv, [row, col])
                        if t % 4 == 0:
                            a0 = a0 + a * bb
                        elif t % 4 == 1:
                            a1 = a1 + a * bb
                        elif t % 4 == 2:
                            a2 = a2 + a * bb
                        else:
                            a3 = a3 + a * bb
                    return (a0, a1, a2, a3)
                a0, a1, a2, a3 = lax.fori_loop(
                    0, ND // 16, dcol, (zero16, zero16, zero16, zero16))
                acc = (a0 + a1) + (a2 + a3)
                off = bp * IDXPAD + koff + k * R + r16 * 16
                tgt = (ujb[pl.ds(off, 16)] * 3
                       + c2b[pl.ds(off, 16)])
                plsc.addupdate_scatter(facc, [tgt], acc)
                return 0
            lax.fori_loop(0, R // 16, rows16, 0)

        issue(0, 0, sem_a)
        @pl.when((b > 0) & (b + 1 < n_b))
        def _():
            for d in _idx_descs(b + 1, 1 - bp, sem_i):
                d.start()

        def pair_body(kk, _):
            k0 = kk * 2
            k1 = k0 + 1

            @pl.when(k1 < jn)
            def _():
                issue(k1, 1, sem_b)
            wait(k0, 0, sem_a)
            compute(k0, 0)

            @pl.when(k0 + 2 < jn)
            def _():
                issue(k0 + 2, 0, sem_a)

            @pl.when(k1 < jn)
            def _():
                wait(k1, 1, sem_b)
                compute(k1, 1)
            return 0
        lax.fori_loop(0, (jn + 1) // 2, pair_body, 0)

        @pl.when(b + 1 < n_b)
        def _():
            for d in _idx_descs(b + 1, 1 - bp, sem_i):
                d.wait()
        return 0
    lax.fori_loop(0, n_b, block_body, 0)

    pltpu.sync_copy(facc, fpart_hbm.at[wid])


def _sc_edge(xd, c0, uj, c2, dEdD, pae, indices):
    mesh = plsc.VectorSubcoreMesh(core_axis_name="c", subcore_axis_name="s",
                                  num_cores=2, num_subcores=16)
    kfn = pl.kernel(
        _sc_body,
        out_type=[
            jax.ShapeDtypeStruct((NW, FPAD), jnp.float32),
            jax.ShapeDtypeStruct((N,), jnp.float32),
        ],
        mesh=mesh,
        compiler_params=pltpu.CompilerParams(needs_layout_passes=False),
        scratch_types=[
            pltpu.VMEM((2 * R, ND), jnp.float32),
            pltpu.VMEM((2 * R, ND), jnp.float32),
            pltpu.VMEM((2 * IDXPAD,), jnp.int32),
            pltpu.VMEM((2 * IDXPAD,), jnp.int32),
            pltpu.VMEM((2 * IDXPAD,), jnp.int32),
            pltpu.VMEM((FPAD,), jnp.float32),
            pltpu.VMEM((EPAD,), jnp.float32),
            pltpu.VMEM((2048,), jnp.float32),
            pltpu.VMEM((2048,), jnp.int32),
            pltpu.SemaphoreType.DMA,
            pltpu.SemaphoreType.DMA,
            pltpu.SemaphoreType.DMA,
        ],
    )
    return kfn(xd, c0, uj, c2, dEdD, pae, indices)


def _combine_body(p_ref, o_ref):
    o_ref[...] = jnp.sum(p_ref[...], axis=0)


def _combine(fpart):
    blk = 3072
    grid = FPAD // blk
    return pl.pallas_call(
        _combine_body,
        grid=(grid,),
        in_specs=[pl.BlockSpec((NW, blk), lambda i: (0, i))],
        out_specs=pl.BlockSpec((blk,), lambda i: (i,)),
        out_shape=jax.ShapeDtypeStruct((FPAD,), jnp.float32),
    )(fpart)


def kernel(x, xd, indices, atoms_per_structure, xd_indx, unique_j,
           W1, b1, W2, b2, W3, b3):
    e, dEdD = _dense(x, W1, b1, W2, b2, W3, b3)
    pae = e.reshape(N)
    c0 = xd_indx[:, 0]
    c2 = xd_indx[:, 2]
    fpart, energy = _sc_edge(xd, c0, unique_j, c2, dEdD, pae, indices)
    forces = _combine(fpart)[:3 * N]
    return (energy, forces)


# speedup vs baseline: 1.0303x; 1.0303x over previous
import jax
import jax.numpy as jnp
from jax import lax
from jax.experimental import pallas as pl
from jax.experimental.pallas import tpu as pltpu
from jax.experimental.pallas import tpu_sc as plsc

N = 10000
ND = 128
M = 480000
H = 64

R = 128
NCH = M // R
NW = 32
FPAD = 30720
EPAD = 10240

DB = 1000


def _dense_body(x_ref, w1_ref, b1_ref, w2_ref, b2_ref, w3t_ref, b3_ref,
                e_ref, g_ref):
    x = x_ref[...]
    w1 = w1_ref[...]
    w2 = w2_ref[...]
    w3t = w3t_ref[...]
    z1 = jnp.dot(x, w1, preferred_element_type=jnp.float32) + b1_ref[...]
    h1 = jnp.maximum(z1, 0.0)
    z2 = jnp.dot(h1, w2, preferred_element_type=jnp.float32) + b2_ref[...]
    h2 = jnp.maximum(z2, 0.0)
    h2r = h2.astype(jnp.bfloat16).astype(jnp.float32)
    w3b = jnp.broadcast_to(w3t, h2.shape
                           ).astype(jnp.bfloat16).astype(jnp.float32)
    e = jnp.sum(h2r * w3b, axis=1, keepdims=True) + b3_ref[0, 0]
    e_ref[...] = e
    g2 = jnp.where(z2 > 0, jnp.broadcast_to(w3t, z2.shape), 0.0)
    g1p = lax.dot_general(g2, w2, (((1,), (1,)), ((), ())),
                          preferred_element_type=jnp.float32)
    g1 = jnp.where(z1 > 0, g1p, 0.0)
    g_ref[...] = lax.dot_general(g1, w1, (((1,), (1,)), ((), ())),
                                 preferred_element_type=jnp.float32)


def _dense(x, W1, b1, W2, b2, W3, b3):
    w3t = W3.reshape(1, H)
    b1r = b1.reshape(1, H)
    b2r = b2.reshape(1, H)
    b3r = b3.reshape(1, 1)
    grid = N // DB
    full = lambda shape: pl.BlockSpec(shape, lambda i: (0, 0))
    return pl.pallas_call(
        _dense_body,
        grid=(grid,),
        in_specs=[
            pl.BlockSpec((DB, ND), lambda i: (i, 0)),
            full((ND, H)), full((1, H)),
            full((H, H)), full((1, H)),
            full((1, H)), full((1, 1)),
        ],
        out_specs=[
            pl.BlockSpec((DB, 1), lambda i: (i, 0)),
            pl.BlockSpec((DB, ND), lambda i: (i, 0)),
        ],
        out_shape=[
            jax.ShapeDtypeStruct((N, 1), jnp.float32),
            jax.ShapeDtypeStruct((N, ND), jnp.float32),
        ],
    )(x, W1, b1r, W2, b2r, w3t, b3r)


BLK = 24
IDXPAD = BLK * R


def _sc_body(xd_hbm, c0_hbm, uj_hbm, c2_hbm, ded_hbm, pae_hbm, idx_hbm,
             fpart_hbm, eout_hbm,
             xdv, dedv, c0b, ujb, c2b, facc, eacc, ebuf, ibuf,
             sem_a, sem_b, sem_i):
    nc = 2
    wid = lax.axis_index("s") * nc + lax.axis_index("c")
    lanes = lax.iota(jnp.int32, 16)

    def zf(i, _):
        for u in range(8):
            facc[pl.ds(i * 128 + u * 16, 16)] = jnp.zeros((16,), jnp.float32)
        return 0
    lax.fori_loop(0, FPAD // 128, zf, 0)

    @pl.when(wid == 0)
    def _energy():
        def ze(i, _):
            for u in range(8):
                eacc[pl.ds(i * 128 + u * 16, 16)] = jnp.zeros((16,),
                                                              jnp.float32)
            return 0
        lax.fori_loop(0, EPAD // 128, ze, 0)

        def echunk(c, _):
            pltpu.sync_copy(pae_hbm.at[pl.ds(c * 2000, 2000)],
                            ebuf.at[pl.ds(0, 2000)])
            pltpu.sync_copy(idx_hbm.at[pl.ds(c * 2000, 2000)],
                            ibuf.at[pl.ds(0, 2000)])

            def ebody(i, _):
                ev = ebuf[pl.ds(i * 16, 16)]
                iv = ibuf[pl.ds(i * 16, 16)]
                plsc.addupdate_scatter(eacc, [iv], ev)
                return 0
            lax.fori_loop(0, 125, ebody, 0)
            return 0
        lax.fori_loop(0, 5, echunk, 0)
        pltpu.sync_copy(eacc.at[pl.ds(0, N)], eout_hbm)

    n_i = jnp.where(wid == 0, 110, 117 + jnp.where(wid < 14, 1, 0))
    start = jnp.where(wid == 0, 0,
                      110 + (wid - 1) * 117 + jnp.minimum(wid - 1, 13))
    n_b = (n_i + BLK - 1) // BLK

    offs = [(lanes + t) & 15 for t in range(16)]
    zero16 = jnp.zeros((16,), jnp.float32)

    def _idx_descs(b, bp, sem):
        base = jnp.minimum((start + b * BLK) * R, M - IDXPAD)
        vb = bp * IDXPAD
        return (
            pltpu.make_async_copy(c0_hbm.at[pl.ds(base, IDXPAD)],
                                  c0b.at[pl.ds(vb, IDXPAD)], sem),
            pltpu.make_async_copy(uj_hbm.at[pl.ds(base, IDXPAD)],
                                  ujb.at[pl.ds(vb, IDXPAD)], sem),
            pltpu.make_async_copy(c2_hbm.at[pl.ds(base, IDXPAD)],
                                  c2b.at[pl.ds(vb, IDXPAD)], sem),
        )

    def _xd_desc(base_rows, k, p, sem):
        return pltpu.make_async_copy(
            xd_hbm.at[pl.ds(base_rows + k * R, R)],
            xdv.at[pl.ds(p * R, R)], sem)

    def _g_desc(bp, koff, k, p, sem):
        return pltpu.make_async_copy(
            ded_hbm.at[c0b.at[pl.ds(bp * IDXPAD + koff + k * R, R)]],
            dedv.at[pl.ds(p * R, R)], sem)

    for d in _idx_descs(0, 0, sem_a):
        d.start()
        d.wait()

    @pl.when(n_b > 1)
    def _():
        for d in _idx_descs(1, 1, sem_i):
            d.start()

    def block_body(b, _):
        j0 = b * BLK
        jn = jnp.minimum(BLK, n_i - j0)
        bp = b % 2
        base_rows = (start + j0) * R
        koff = base_rows - jnp.minimum(base_rows, M - IDXPAD)

        def issue(k, p, sem):
            _xd_desc(base_rows, k, p, sem).start()
            _g_desc(bp, koff, k, p, sem).start()

        def wait(k, p, sem):
            _xd_desc(base_rows, k, p, sem).wait()
            _g_desc(bp, koff, k, p, sem).wait()

        def compute(k, p):
            def rows16(r16, _):
                row = p * R + r16 * 16 + lanes

                def dcol(j, accs):
                    a0, a1, a2, a3 = accs
                    dvec = jnp.full((16,), j * 16, jnp.int32)
                    for t in range(16):
                        col = offs[t] + dvec
                        a = plsc.load_gather(xdv, [row, col])
                        bb = plsc.load_gather(dedv, [row, col])
                        if t % 4 == 0:
                            a0 = a0 + a * bb
                        elif t % 4 == 1:
                            a1 = a1 + a * bb
                        elif t % 4 == 2:
                            a2 = a2 + a * bb
                        else:
                            a3 = a3 + a * bb
                    return (a0, a1, a2, a3)
                a0, a1, a2, a3 = lax.fori_loop(
                    0, ND // 16, dcol, (zero16, zero16, zero16, zero16))
                acc = (a0 + a1) + (a2 + a3)
                off = bp * IDXPAD + koff + k * R + r16 * 16
                tgt = (ujb[pl.ds(off, 16)] * 3
                       + c2b[pl.ds(off, 16)])
                plsc.addupdate_scatter(facc, [tgt], acc)
                return 0
            lax.fori_loop(0, R // 16, rows16, 0)

        issue(0, 0, sem_a)
        @pl.when((b > 0) & (b + 1 < n_b))
        def _():
            for d in _idx_descs(b + 1, 1 - bp, sem_i):
                d.start()

        def pair_body(kk, _):
            k0 = kk * 2
            k1 = k0 + 1

            @pl.when(k1 < jn)
            def _():
                issue(k1, 1, sem_b)
            wait(k0, 0, sem_a)
            compute(k0, 0)

            @pl.when(k0 + 2 < jn)
            def _():
                issue(k0 + 2, 0, sem_a)

            @pl.when(k1 < jn)
            def _():
                wait(k1, 1, sem_b)
                compute(k1, 1)
            return 0
        lax.fori_loop(0, (jn + 1) // 2, pair_body, 0)

        @pl.when(b + 1 < n_b)
        def _():
            for d in _idx_descs(b + 1, 1 - bp, sem_i):
                d.wait()
        return 0
    lax.fori_loop(0, n_b, block_body, 0)

    pltpu.sync_copy(facc, fpart_hbm.at[wid])


def _sc_edge(xd, c0, uj, c2, dEdD, pae, indices):
    mesh = plsc.VectorSubcoreMesh(core_axis_name="c", subcore_axis_name="s",
                                  num_cores=2, num_subcores=16)
    kfn = pl.kernel(
        _sc_body,
        out_type=[
            jax.ShapeDtypeStruct((NW, FPAD), jnp.float32),
            jax.ShapeDtypeStruct((N,), jnp.float32),
        ],
        mesh=mesh,
        compiler_params=pltpu.CompilerParams(needs_layout_passes=False),
        scratch_types=[
            pltpu.VMEM((2 * R, ND), jnp.float32),
            pltpu.VMEM((2 * R, ND), jnp.float32),
            pltpu.VMEM((2 * IDXPAD,), jnp.int32),
            pltpu.VMEM((2 * IDXPAD,), jnp.int32),
            pltpu.VMEM((2 * IDXPAD,), jnp.int32),
            pltpu.VMEM((FPAD,), jnp.float32),
            pltpu.VMEM((EPAD,), jnp.float32),
            pltpu.VMEM((2048,), jnp.float32),
            pltpu.VMEM((2048,), jnp.int32),
            pltpu.SemaphoreType.DMA,
            pltpu.SemaphoreType.DMA,
            pltpu.SemaphoreType.DMA,
        ],
    )
    return kfn(xd, c0, uj, c2, dEdD, pae, indices)


def _combine_body(p_ref, o_ref):
    o_ref[...] = jnp.sum(p_ref[...], axis=0)


def _combine(fpart):
    blk = 3072
    grid = FPAD // blk
    return pl.pallas_call(
        _combine_body,
        grid=(grid,),
        in_specs=[pl.BlockSpec((NW, blk), lambda i: (0, i))],
        out_specs=pl.BlockSpec((blk,), lambda i: (i,)),
        out_shape=jax.ShapeDtypeStruct((FPAD,), jnp.float32),
    )(fpart)


def kernel(x, xd, indices, atoms_per_structure, xd_indx, unique_j,
           W1, b1, W2, b2, W3, b3):
    e, dEdD = _dense(x, W1, b1, W2, b2, W3, b3)
    pae = e.reshape(N)
    c0 = xd_indx[:, 0]
    c2 = xd_indx[:, 2]
    fpart, energy = _sc_edge(xd, c0, unique_j, c2, dEdD, pae, indices)
    forces = _combine(fpart)[:3 * N]
    return (energy, forces)


# speedup vs baseline: 1.0698x; 1.0384x over previous
import jax
import jax.numpy as jnp
from jax import lax
from jax.experimental import pallas as pl
from jax.experimental.pallas import tpu as pltpu
from jax.experimental.pallas import tpu_sc as plsc

N = 10000
ND = 128
M = 480000
H = 64

R = 128
NCH = M // R
NW = 32
FPAD = 30720
EPAD = 10240

DB = 1000


def _dense_body(x_ref, w1_ref, b1_ref, w2_ref, b2_ref, w3t_ref, b3_ref,
                e_ref, g_ref):
    x = x_ref[...]
    w1 = w1_ref[...]
    w2 = w2_ref[...]
    w3t = w3t_ref[...]
    z1 = jnp.dot(x, w1, preferred_element_type=jnp.float32) + b1_ref[...]
    h1 = jnp.maximum(z1, 0.0)
    z2 = jnp.dot(h1, w2, preferred_element_type=jnp.float32) + b2_ref[...]
    h2 = jnp.maximum(z2, 0.0)
    h2r = h2.astype(jnp.bfloat16).astype(jnp.float32)
    w3b = jnp.broadcast_to(w3t, h2.shape
                           ).astype(jnp.bfloat16).astype(jnp.float32)
    e = jnp.sum(h2r * w3b, axis=1, keepdims=True) + b3_ref[0, 0]
    e_ref[...] = e
    g2 = jnp.where(z2 > 0, jnp.broadcast_to(w3t, z2.shape), 0.0)
    g1p = lax.dot_general(g2, w2, (((1,), (1,)), ((), ())),
                          preferred_element_type=jnp.float32)
    g1 = jnp.where(z1 > 0, g1p, 0.0)
    g_ref[...] = lax.dot_general(g1, w1, (((1,), (1,)), ((), ())),
                                 preferred_element_type=jnp.float32)


def _dense(x, W1, b1, W2, b2, W3, b3):
    w3t = W3.reshape(1, H)
    b1r = b1.reshape(1, H)
    b2r = b2.reshape(1, H)
    b3r = b3.reshape(1, 1)
    grid = N // DB
    full = lambda shape: pl.BlockSpec(shape, lambda i: (0, 0))
    return pl.pallas_call(
        _dense_body,
        grid=(grid,),
        in_specs=[
            pl.BlockSpec((DB, ND), lambda i: (i, 0)),
            full((ND, H)), full((1, H)),
            full((H, H)), full((1, H)),
            full((1, H)), full((1, 1)),
        ],
        out_specs=[
            pl.BlockSpec((DB, 1), lambda i: (i, 0)),
            pl.BlockSpec((DB, ND), lambda i: (i, 0)),
        ],
        out_shape=[
            jax.ShapeDtypeStruct((N, 1), jnp.float32),
            jax.ShapeDtypeStruct((N, ND), jnp.float32),
        ],
    )(x, W1, b1r, W2, b2r, w3t, b3r)


BLK = 24
IDXPAD = BLK * R


def _sc_body(xd_hbm, c0_hbm, uj_hbm, c2_hbm, ded_hbm, pae_hbm, idx_hbm,
             fpart_hbm, eout_hbm,
             xdv, dedv, c0b, ujb, c2b, facc, eacc, ebuf, ibuf,
             sem_a, sem_b, sem_i):
    nc = 2
    wid = lax.axis_index("s") * nc + lax.axis_index("c")
    lanes = lax.iota(jnp.int32, 16)

    def zf(i, _):
        for u in range(8):
            facc[pl.ds(i * 128 + u * 16, 16)] = jnp.zeros((16,), jnp.float32)
        return 0
    lax.fori_loop(0, FPAD // 128, zf, 0)

    @pl.when(wid == 0)
    def _energy():
        def ze(i, _):
            for u in range(8):
                eacc[pl.ds(i * 128 + u * 16, 16)] = jnp.zeros((16,),
                                                              jnp.float32)
            return 0
        lax.fori_loop(0, EPAD // 128, ze, 0)

        def echunk(c, _):
            pltpu.sync_copy(pae_hbm.at[pl.ds(c * 2000, 2000)],
                            ebuf.at[pl.ds(0, 2000)])
            pltpu.sync_copy(idx_hbm.at[pl.ds(c * 2000, 2000)],
                            ibuf.at[pl.ds(0, 2000)])

            def ebody(i, _):
                ev = ebuf[pl.ds(i * 16, 16)]
                iv = ibuf[pl.ds(i * 16, 16)]
                plsc.addupdate_scatter(eacc, [iv], ev)
                return 0
            lax.fori_loop(0, 125, ebody, 0)
            return 0
        lax.fori_loop(0, 5, echunk, 0)
        pltpu.sync_copy(eacc.at[pl.ds(0, N)], eout_hbm)

    n_i = jnp.where(wid == 0, 110, 117 + jnp.where(wid < 14, 1, 0))
    start = jnp.where(wid == 0, 0,
                      110 + (wid - 1) * 117 + jnp.minimum(wid - 1, 13))
    n_b = (n_i + BLK - 1) // BLK

    offs = [(lanes + t) & 15 for t in range(16)]
    zero16 = jnp.zeros((16,), jnp.float32)

    def _idx_descs(b, bp, sem):
        base = jnp.minimum((start + b * BLK) * R, M - IDXPAD)
        vb = bp * IDXPAD
        return (
            pltpu.make_async_copy(c0_hbm.at[pl.ds(base, IDXPAD)],
                                  c0b.at[pl.ds(vb, IDXPAD)], sem),
            pltpu.make_async_copy(uj_hbm.at[pl.ds(base, IDXPAD)],
                                  ujb.at[pl.ds(vb, IDXPAD)], sem),
            pltpu.make_async_copy(c2_hbm.at[pl.ds(base, IDXPAD)],
                                  c2b.at[pl.ds(vb, IDXPAD)], sem),
        )

    def _xd_desc(base_rows, k, p, sem):
        return pltpu.make_async_copy(
            xd_hbm.at[pl.ds(base_rows + k * R, R)],
            xdv.at[pl.ds(p * R, R)], sem)

    def _g_desc2(soff, p, sem):
        return pltpu.make_async_copy(
            ded_hbm.at[c0b.at[pl.ds(soff, R)]],
            dedv.at[pl.ds(p * R, R)], sem)

    for d in _idx_descs(0, 0, sem_a):
        d.start()
        d.wait()

    @pl.when(n_b > 1)
    def _():
        for d in _idx_descs(1, 1, sem_i):
            d.start()

    def _soff(k):
        b = k // BLK
        base_rows = (start + b * BLK) * R
        koff = base_rows - jnp.minimum(base_rows, M - IDXPAD)
        return b, (b % 2) * IDXPAD + koff + (k - b * BLK) * R

    def issue(k, p, sem):
        b, soff = _soff(k)

        @pl.when((k > 0) & (k % BLK == 0))
        def _():
            for d in _idx_descs(b, b % 2, sem_i):
                d.wait()
        _xd_desc((start + k) * R, 0, p, sem).start()
        _g_desc2(soff, p, sem).start()

    def wait(k, p, sem):
        _, soff = _soff(k)
        _xd_desc((start + k) * R, 0, p, sem).wait()
        _g_desc2(soff, p, sem).wait()

    def maybe_prefetch(k):
        @pl.when(((k + 1) % BLK == 0) & ((k + 1) // BLK + 1 < n_b))
        def _():
            bb = (k + 1) // BLK + 1
            for d in _idx_descs(bb, bb % 2, sem_i):
                d.start()

    def compute(k, p):
        _, soff = _soff(k)

        def rows16(r16, _):
            row = p * R + r16 * 16 + lanes

            def dcol(j, accs):
                a0, a1, a2, a3 = accs
                dvec = jnp.full((16,), j * 16, jnp.int32)
                for t in range(16):
                    col = offs[t] + dvec
                    a = plsc.load_gather(xdv, [row, col])
                    bb = plsc.load_gather(dedv, [row, col])
                    if t % 4 == 0:
                        a0 = a0 + a * bb
                    elif t % 4 == 1:
                        a1 = a1 + a * bb
                    elif t % 4 == 2:
                        a2 = a2 + a * bb
                    else:
                        a3 = a3 + a * bb
                return (a0, a1, a2, a3)
            a0, a1, a2, a3 = lax.fori_loop(
                0, ND // 16, dcol, (zero16, zero16, zero16, zero16))
            acc = (a0 + a1) + (a2 + a3)
            off = soff + r16 * 16
            tgt = ujb[pl.ds(off, 16)] * 3 + c2b[pl.ds(off, 16)]
            plsc.addupdate_scatter(facc, [tgt], acc)
            return 0
        lax.fori_loop(0, R // 16, rows16, 0)

    issue(0, 0, sem_a)

    def pair_body(kk, _):
        k0 = kk * 2
        k1 = k0 + 1

        @pl.when(k1 < n_i)
        def _():
            issue(k1, 1, sem_b)
        wait(k0, 0, sem_a)
        compute(k0, 0)
        maybe_prefetch(k0)

        @pl.when(k0 + 2 < n_i)
        def _():
            issue(k0 + 2, 0, sem_a)

        @pl.when(k1 < n_i)
        def _():
            wait(k1, 1, sem_b)
            compute(k1, 1)
            maybe_prefetch(k1)
        return 0
    lax.fori_loop(0, (n_i + 1) // 2, pair_body, 0)

    pltpu.sync_copy(facc, fpart_hbm.at[wid])


def _sc_edge(xd, c0, uj, c2, dEdD, pae, indices):
    mesh = plsc.VectorSubcoreMesh(core_axis_name="c", subcore_axis_name="s",
                                  num_cores=2, num_subcores=16)
    kfn = pl.kernel(
        _sc_body,
        out_type=[
            jax.ShapeDtypeStruct((NW, FPAD), jnp.float32),
            jax.ShapeDtypeStruct((N,), jnp.float32),
        ],
        mesh=mesh,
        compiler_params=pltpu.CompilerParams(needs_layout_passes=False),
        scratch_types=[
            pltpu.VMEM((2 * R, ND), jnp.float32),
            pltpu.VMEM((2 * R, ND), jnp.float32),
            pltpu.VMEM((2 * IDXPAD,), jnp.int32),
            pltpu.VMEM((2 * IDXPAD,), jnp.int32),
            pltpu.VMEM((2 * IDXPAD,), jnp.int32),
            pltpu.VMEM((FPAD,), jnp.float32),
            pltpu.VMEM((EPAD,), jnp.float32),
            pltpu.VMEM((2048,), jnp.float32),
            pltpu.VMEM((2048,), jnp.int32),
            pltpu.SemaphoreType.DMA,
            pltpu.SemaphoreType.DMA,
            pltpu.SemaphoreType.DMA,
        ],
    )
    return kfn(xd, c0, uj, c2, dEdD, pae, indices)


def _combine_body(p_ref, o_ref):
    o_ref[...] = jnp.sum(p_ref[...], axis=0)


def _combine(fpart):
    blk = 3072
    grid = FPAD // blk
    return pl.pallas_call(
        _combine_body,
        grid=(grid,),
        in_specs=[pl.BlockSpec((NW, blk), lambda i: (0, i))],
        out_specs=pl.BlockSpec((blk,), lambda i: (i,)),
        out_shape=jax.ShapeDtypeStruct((FPAD,), jnp.float32),
    )(fpart)


def kernel(x, xd, indices, atoms_per_structure, xd_indx, unique_j,
           W1, b1, W2, b2, W3, b3):
    e, dEdD = _dense(x, W1, b1, W2, b2, W3, b3)
    pae = e.reshape(N)
    c0 = xd_indx[:, 0]
    c2 = xd_indx[:, 2]
    fpart, energy = _sc_edge(xd, c0, unique_j, c2, dEdD, pae, indices)
    forces = _combine(fpart)[:3 * N]
    return (energy, forces)
